# baseline (speedup 1.0000x reference)
import jax
import jax.numpy as jnp
from jax import lax
from jax.experimental import pallas as pl

_D = 10
_OUT = 224
_B = 2
_KMAX = 384


def _resize_matrix(n_in: int) -> jnp.ndarray:
    eye = jnp.eye(n_in, dtype=jnp.float32)
    return jax.image.resize(eye, (_OUT, n_in), method="bilinear")


def _rc_kernel(d_ref, f1_ref, f2_ref, f3_ref,
               a1_ref, a1t_ref, a2_ref, a2t_ref, a3_ref, a3t_ref,
               out_ref):
    depth = d_ref[...]
    d_min = jnp.min(depth)
    d_max = jnp.max(depth)
    step = (d_max - d_min) / _D

    masks = []
    counts = []
    for dd in range(_D):
        lo = d_min + dd * step
        if dd < _D - 1:
            hi = d_min + (dd + 1) * step
            m = jnp.logical_and(depth >= lo, depth < hi)
        else:
            m = depth >= lo
        mf = m.astype(jnp.float32)
        masks.append(mf)
        counts.append(jnp.maximum(jnp.sum(mf), 1e-6))
    mfull = jnp.stack(masks, axis=0)
    mflat = mfull.reshape(_D * _B * _OUT, _OUT)

    out_ref[...] = jnp.zeros(out_ref.shape, dtype=jnp.float32)

    for l, (f_ref, a_ref, at_ref) in enumerate(
            ((f1_ref, a1_ref, a1t_ref),
             (f2_ref, a2_ref, a2t_ref),
             (f3_ref, a3_ref, a3t_ref))):
        f = f_ref[...]
        a = a_ref[...]
        at = at_ref[...]
        c_dim, h = f.shape[1], f.shape[2]
        t = jnp.dot(mflat, a, preferred_element_type=jnp.float32)
        t = t.reshape(_D, _B, _OUT, h)
        w = lax.dot_general(
            at, t, (((1,), (2,)), ((), ())),
            preferred_element_type=jnp.float32)
        cols = []
        for dd in range(_D):
            acc = None
            for b in range(_B):
                s = jnp.sum(f[b] * w[:, dd, b, :][None, :, :], axis=(1, 2))
                acc = s if acc is None else acc + s
            cols.append((acc / counts[dd])[:, None])
        out_ref[l, :c_dim, :] = jnp.concatenate(cols, axis=1)


def kernel(fmap1, fmap2, fmap3, depths):
    d = depths[:, 0]
    mats = []
    for f in (fmap1, fmap2, fmap3):
        a = _resize_matrix(f.shape[2])
        mats.extend([a, a.T])
    out = pl.pallas_call(
        _rc_kernel,
        out_shape=jax.ShapeDtypeStruct((3, _KMAX, _D), jnp.float32),
    )(d, fmap1, fmap2, fmap3, *mats)
    return out


# speedup vs baseline: 1.1519x; 1.1519x over previous
import numpy as np

import jax
import jax.numpy as jnp
from jax.experimental import pallas as pl

_D = 10
_OUT = 224
_B = 2
_KMAX = 384


def _resize_matrix(n_in: int) -> np.ndarray:
    inv = n_in / _OUT
    sample_f = (np.arange(_OUT, dtype=np.float32) + 0.5) * np.float32(inv) - 0.5
    x = np.abs(sample_f[None, :] - np.arange(n_in, dtype=np.float32)[:, None])
    w = np.maximum(0.0, 1.0 - x).astype(np.float32)
    total = w.sum(axis=0, keepdims=True)
    w = np.where(np.abs(total) > 1e-6, w / total, 0.0).astype(np.float32)
    return np.ascontiguousarray(w.T)


def _rc_kernel(d_ref, f1_ref, f2_ref, f3_ref,
               a1_ref, a1t_ref, a2_ref, a2t_ref, a3_ref, a3t_ref,
               out_ref):
    depth = d_ref[...]
    d_min = jnp.min(depth)
    d_max = jnp.max(depth)
    step = (d_max - d_min) / _D

    masks = []
    counts = []
    for dd in range(_D):
        lo = d_min + dd * step
        if dd < _D - 1:
            hi = d_min + (dd + 1) * step
            m = jnp.logical_and(depth >= lo, depth < hi)
        else:
            m = depth >= lo
        mf = m.astype(jnp.float32)
        masks.append(mf)
        counts.append(jnp.maximum(jnp.sum(mf), 1e-6))
    mfull = jnp.stack(masks, axis=0)
    mflat = mfull.reshape(_D * _B * _OUT, _OUT)

    out_ref[...] = jnp.zeros(out_ref.shape, dtype=jnp.float32)

    for l, (f_ref, a_ref, at_ref) in enumerate(
            ((f1_ref, a1_ref, a1t_ref),
             (f2_ref, a2_ref, a2t_ref),
             (f3_ref, a3_ref, a3t_ref))):
        f = f_ref[...]
        a = a_ref[...]
        at = at_ref[...]
        c_dim, h = f.shape[1], f.shape[2]
        t = jnp.dot(mflat, a, preferred_element_type=jnp.float32)
        t = t.reshape(_D, _B, _OUT, h)
        cols = []
        for dd in range(_D):
            acc = None
            for b in range(_B):
                w_db = jnp.dot(at, t[dd, b],
                               preferred_element_type=jnp.float32)
                s = jnp.sum(f[b] * w_db[None, :, :], axis=(1, 2))
                acc = s if acc is None else acc + s
            cols.append((acc / counts[dd])[:, None])
        out_ref[l, :c_dim, :] = jnp.concatenate(cols, axis=1)


def kernel(fmap1, fmap2, fmap3, depths):
    d = depths[:, 0]
    mats = []
    for f in (fmap1, fmap2, fmap3):
        a = _resize_matrix(f.shape[2])
        mats.extend([a, np.ascontiguousarray(a.T)])
    out = pl.pallas_call(
        _rc_kernel,
        out_shape=jax.ShapeDtypeStruct((3, _KMAX, _D), jnp.float32),
    )(d, fmap1, fmap2, fmap3, *mats)
    return out


# speedup vs baseline: 1.9666x; 1.7073x over previous
import numpy as np

import jax
import jax.numpy as jnp
from jax.experimental import pallas as pl

_D = 10
_OUT = 224
_B = 2
_KMAX = 384


def _resize_matrix(n_in: int) -> np.ndarray:
    inv = n_in / _OUT
    sample_f = (np.arange(_OUT, dtype=np.float32) + 0.5) * np.float32(inv) - 0.5
    x = np.abs(sample_f[None, :] - np.arange(n_in, dtype=np.float32)[:, None])
    w = np.maximum(0.0, 1.0 - x).astype(np.float32)
    total = w.sum(axis=0, keepdims=True)
    w = np.where(np.abs(total) > 1e-6, w / total, 0.0).astype(np.float32)
    return np.ascontiguousarray(w.T)


def _weights_kernel(d_ref, a1_ref, a1t_ref, a2_ref, a2t_ref, a3_ref, a3t_ref,
                    w1_ref, w2_ref, w3_ref, ic_ref):
    depth = d_ref[...]
    d_min = jnp.min(depth)
    d_max = jnp.max(depth)
    step = (d_max - d_min) / _D

    masks = []
    inv_counts = []
    for dd in range(_D):
        lo = d_min + dd * step
        if dd < _D - 1:
            hi = d_min + (dd + 1) * step
            m = jnp.logical_and(depth >= lo, depth < hi)
        else:
            m = depth >= lo
        mf = m.astype(jnp.float32)
        masks.append(mf)
        inv_counts.append((1.0 / jnp.maximum(jnp.sum(mf), 1e-6))[None, None])
    mfull = jnp.stack(masks, axis=0)
    mflat = mfull.reshape(_D * _B * _OUT, _OUT)
    ic_ref[...] = jnp.concatenate(inv_counts, axis=1)

    for (a_ref, at_ref, w_ref) in ((a1_ref, a1t_ref, w1_ref),
                                   (a2_ref, a2t_ref, w2_ref),
                                   (a3_ref, a3t_ref, w3_ref)):
        a = a_ref[...]
        at = at_ref[...]
        h = at.shape[0]
        t = jnp.dot(mflat, a, preferred_element_type=jnp.float32)
        t = t.reshape(_D, _B, _OUT, h)
        for dd in range(_D):
            for b in range(_B):
                w_ref[dd, b] = jnp.dot(at, t[dd, b],
                                       preferred_element_type=jnp.float32)


def _contract_kernel(ff1_ref, ff2_ref, ff3_ref, w1_ref, w2_ref, w3_ref,
                     ic_ref, out_ref):
    inv_counts = ic_ref[...]
    out_ref[...] = jnp.zeros(out_ref.shape, dtype=jnp.float32)
    for l, (ff_ref, w_ref) in enumerate(((ff1_ref, w1_ref),
                                         (ff2_ref, w2_ref),
                                         (ff3_ref, w3_ref))):
        c_dim = ff_ref.shape[0] // _B
        acc = None
        for b in range(_B):
            s = jax.lax.dot_general(
                ff_ref[b * c_dim:(b + 1) * c_dim, :], w_ref[b],
                (((1,), (1,)), ((), ())),
                preferred_element_type=jnp.float32)
            acc = s if acc is None else acc + s
        out_ref[l, :c_dim, :] = acc * inv_counts


def kernel(fmap1, fmap2, fmap3, depths):
    d = depths[:, 0]
    mats = []
    w_shapes = []
    for f in (fmap1, fmap2, fmap3):
        a = _resize_matrix(f.shape[2])
        mats.extend([a, np.ascontiguousarray(a.T)])
        w_shapes.append(
            jax.ShapeDtypeStruct((_D, _B, f.shape[2], f.shape[3]), jnp.float32))

    w1, w2, w3, ic = pl.pallas_call(
        _weights_kernel,
        out_shape=(w_shapes[0], w_shapes[1], w_shapes[2],
                   jax.ShapeDtypeStruct((1, _D), jnp.float32)),
    )(d, *mats)

    ffs = [f.reshape(f.shape[0] * f.shape[1], f.shape[2] * f.shape[3])
           for f in (fmap1, fmap2, fmap3)]
    wfs = [jnp.transpose(w, (1, 0, 2, 3)).reshape(
               _B, _D, w.shape[2] * w.shape[3])
           for w in (w1, w2, w3)]

    out = pl.pallas_call(
        _contract_kernel,
        out_shape=jax.ShapeDtypeStruct((3, _KMAX, _D), jnp.float32),
    )(ffs[0], ffs[1], ffs[2], wfs[0], wfs[1], wfs[2], ic)
    return out
